# baseline (speedup 1.0000x reference)
import functools

import jax
import jax.numpy as jnp
from jax import lax
from jax.experimental import pallas as pl
from jax.experimental.pallas import tpu as pltpu
from jax.experimental.pallas import tpu_sc as plsc

NW = 32
NSUB = 16
LANES = 16
D = 128
DH = 64
ECH = 128
GSZ = 80
NB = 4


def _sc_gather_deg(npad, cep):
    rpt = npad // NW
    nchunk = rpt // GSZ
    nps = npad // NSUB
    hc = cep // 2
    mesh = plsc.VectorSubcoreMesh(core_axis_name="c", subcore_axis_name="s")

    @functools.partial(
        pl.kernel,
        out_type=(
            jax.ShapeDtypeStruct((npad, D), jnp.float32),
            jax.ShapeDtypeStruct((2, npad), jnp.float32),
        ),
        mesh=mesh,
        scratch_types=[
            pltpu.VMEM((nchunk, GSZ), jnp.int32),
            pltpu.VMEM((rpt, D), jnp.float32),
            pltpu.VMEM((hc, ECH), jnp.int32),
            pltpu.VMEM((ECH,), jnp.float32),
            pltpu.VMEM_SHARED((npad,), jnp.float32),
            pltpu.SemaphoreType.DMA,
        ],
    )
    def k(fi_hbm, dst_hbm, deginit_hbm, emb_hbm, xg_out, degp_out,
          fi_v, rows_v, dst_v, ones_v, deg_sp, sem):
        c = lax.axis_index("c")
        s = lax.axis_index("s")
        wid = s * 2 + c

        @pl.when(s == 0)
        def _():
            pltpu.sync_copy(deginit_hbm.at[c], deg_sp)

        def setones(i, carry):
            ones_v[pl.ds(i * LANES, LANES)] = jnp.ones((LANES,), jnp.float32)
            return carry
        lax.fori_loop(0, ECH // LANES, setones, 0)

        pltpu.sync_copy(fi_hbm.at[wid], fi_v)
        pltpu.sync_copy(dst_hbm.at[s, pl.ds(c * hc, hc)], dst_v)
        plsc.subcore_barrier()

        def degbody(j, carry):
            pltpu.sync_copy(ones_v, deg_sp.at[dst_v.at[j]], add=True)
            return carry
        lax.fori_loop(0, hc, degbody, 0)

        for g in range(nchunk):
            pltpu.async_copy(
                emb_hbm.at[fi_v.at[g]],
                rows_v.at[pl.ds(g * GSZ, GSZ)], sem).wait()
        pltpu.sync_copy(rows_v, xg_out.at[pl.ds(wid * rpt, rpt)])

        plsc.subcore_barrier()
        pltpu.sync_copy(deg_sp.at[pl.ds(s * nps, nps)],
                        degp_out.at[c, pl.ds(s * nps, nps)])

    return k


def _sc_edge_agg(npad, ce, cpad):
    nps = npad // NSUB
    mesh = plsc.VectorSubcoreMesh(core_axis_name="c", subcore_axis_name="s")

    @functools.partial(
        pl.kernel,
        out_type=jax.ShapeDtypeStruct((2, npad, DH), jnp.float32),
        mesh=mesh,
        scratch_types=[
            pltpu.VMEM((cpad, ECH), jnp.int32),
            pltpu.VMEM((cpad, ECH), jnp.int32),
            pltpu.VMEM((NB, ECH, DH), jnp.float32),
            pltpu.VMEM_SHARED((npad, DH), jnp.float32),
            pltpu.SemaphoreType.DMA((NB,)),
        ],
        compiler_params=pltpu.CompilerParams(use_tc_tiling_on_sc=False),
    )
    def k(y_hbm, src_hbm, dst_hbm, tout, src_v, dst_v, rows_v, t_sp, gsem):
        c = lax.axis_index("c")
        s = lax.axis_index("s")

        def zrow(i, carry):
            def zcol(j, carry2):
                rows_v[0, i, pl.ds(j * LANES, LANES)] = (
                    jnp.zeros((LANES,), jnp.float32))
                return carry2
            return lax.fori_loop(0, DH // LANES, zcol, carry)
        lax.fori_loop(0, ECH, zrow, 0)
        for r in range(nps // ECH):
            pltpu.sync_copy(rows_v.at[0],
                            t_sp.at[pl.ds(s * nps + r * ECH, ECH)])

        pltpu.sync_copy(src_hbm.at[s], src_v)
        pltpu.sync_copy(dst_hbm.at[s], dst_v)
        plsc.subcore_barrier()

        yh = y_hbm.at[c]

        for b in range(NB):
            pltpu.async_copy(yh.at[src_v.at[b]], rows_v.at[b], gsem.at[b])

        def wait_gather(b):
            pltpu.make_async_copy(yh.at[pl.ds(0, ECH)],
                                  rows_v.at[b], gsem.at[b]).wait()

        def group(g, carry):
            for b in range(NB):
                j = g * NB + b
                wait_gather(b)
                pltpu.sync_copy(rows_v.at[b], t_sp.at[dst_v.at[j]],
                                add=True)
                pltpu.async_copy(yh.at[src_v.at[j + NB]], rows_v.at[b],
                                 gsem.at[b])
            return carry
        lax.fori_loop(0, ce // NB, group, 0)
        for b in range(NB):
            wait_gather(b)

        plsc.subcore_barrier()
        pltpu.sync_copy(t_sp.at[pl.ds(s * nps, nps)],
                        tout.at[c, pl.ds(s * nps, nps)])

    return k


def _tc1(npad, blk):
    grid = npad // blk

    def body(xg, fw, degp, w1, y1, dis_out):
        x = xg[...] * fw[...]
        nrm = jnp.sqrt(jnp.sum(x * x, axis=1, keepdims=True))
        x = x / jnp.maximum(nrm, 1e-12)
        deg = degp[..., 0:1] + degp[..., 1:2]
        dis = lax.rsqrt(deg)
        y = jnp.dot(x, w1[...], preferred_element_type=jnp.float32) * dis
        y1[0] = y[:, :DH]
        y1[1] = y[:, DH:]
        dis_out[...] = dis

    return pl.pallas_call(
        body,
        grid=(grid,),
        in_specs=[
            pl.BlockSpec((blk, D), lambda i: (i, 0)),
            pl.BlockSpec((blk, 1), lambda i: (i, 0)),
            pl.BlockSpec((blk, 2), lambda i: (i, 0)),
            pl.BlockSpec((D, D), lambda i: (0, 0)),
        ],
        out_specs=[
            pl.BlockSpec((2, blk, DH), lambda i: (0, i, 0)),
            pl.BlockSpec((blk, 1), lambda i: (i, 0)),
        ],
        out_shape=[
            jax.ShapeDtypeStruct((2, npad, DH), jnp.float32),
            jax.ShapeDtypeStruct((npad, 1), jnp.float32),
        ],
    )


def _tc2(npad, blk):
    grid = npad // blk

    def body(tp, y1, dis, b1, wcat, y2):
        agg = jnp.concatenate([tp[0] + y1[0], tp[1] + y1[1]], axis=1)
        h = jnp.maximum(dis[...] * agg + b1[...], 0.0)
        y = jnp.dot(h, wcat[...],
                    preferred_element_type=jnp.float32) * dis[...]
        y2[0] = y[:, :DH]
        y2[1] = y[:, DH:]

    return pl.pallas_call(
        body,
        grid=(grid,),
        in_specs=[
            pl.BlockSpec((2, blk, DH), lambda i: (0, i, 0)),
            pl.BlockSpec((2, blk, DH), lambda i: (0, i, 0)),
            pl.BlockSpec((blk, 1), lambda i: (i, 0)),
            pl.BlockSpec((1, D), lambda i: (0, 0)),
            pl.BlockSpec((D, D), lambda i: (0, 0)),
        ],
        out_specs=pl.BlockSpec((2, blk, DH), lambda i: (0, i, 0)),
        out_shape=jax.ShapeDtypeStruct((2, npad, DH), jnp.float32),
    )


def _tc3(npad, blk, dout):
    grid = npad // blk

    def body(tp, y2, dis, bmu, bls, noise, z):
        mu = dis[...] * (tp[0] + y2[0]) + bmu[...]
        ls = dis[...] * (tp[1] + y2[1]) + bls[...]
        z[...] = mu + noise[...] * jnp.exp(ls)

    return pl.pallas_call(
        body,
        grid=(grid,),
        in_specs=[
            pl.BlockSpec((2, blk, DH), lambda i: (0, i, 0)),
            pl.BlockSpec((2, blk, DH), lambda i: (0, i, 0)),
            pl.BlockSpec((blk, 1), lambda i: (i, 0)),
            pl.BlockSpec((1, dout), lambda i: (0, 0)),
            pl.BlockSpec((1, dout), lambda i: (0, 0)),
            pl.BlockSpec((blk, dout), lambda i: (i, 0)),
        ],
        out_specs=pl.BlockSpec((blk, dout), lambda i: (i, 0)),
        out_shape=jax.ShapeDtypeStruct((npad, dout), jnp.float32),
    )


def kernel(feature_indices, feature_offsets, feature_weights, edge_index,
           emb_table, W1, b1, W_mu, b_mu, W_ls, b_ls, noise):
    n = feature_offsets.shape[0]
    e = edge_index.shape[1]
    dout = W_mu.shape[1]

    npad = ((n + NW * GSZ - 1) // (NW * GSZ)) * (NW * GSZ)
    ce = (e + NSUB * ECH - 1) // (NSUB * ECH)
    ce = ((ce + NB - 1) // NB) * NB
    epad = NSUB * ce * ECH
    rpt = npad // NW

    fi = jnp.zeros((npad,), jnp.int32).at[:n].set(feature_indices)
    fi3 = fi.reshape(NW, rpt // GSZ, GSZ)
    fw = jnp.zeros((npad, 1), jnp.float32).at[:n, 0].set(feature_weights)
    src = jnp.full((epad,), n, jnp.int32).at[:e].set(edge_index[0])
    dst = jnp.full((epad,), n, jnp.int32).at[:e].set(edge_index[1])
    cpad = ((ce + NB + 15) // 16) * 16
    drain = jnp.full((NSUB, cpad - ce, ECH), n, jnp.int32)
    src3 = jnp.concatenate([src.reshape(NSUB, ce, ECH), drain], axis=1)
    dst3 = jnp.concatenate([dst.reshape(NSUB, ce, ECH), drain], axis=1)
    deginit = jnp.stack(
        [jnp.ones((npad,), jnp.float32), jnp.zeros((npad,), jnp.float32)])
    wcat = jnp.concatenate([W_mu, W_ls], axis=1)
    noise_p = jnp.zeros((npad, dout), jnp.float32).at[:n].set(noise)

    xg, degp = _sc_gather_deg(npad, cpad)(fi3, dst3, deginit, emb_table)
    degp_t = degp.T

    blk = 1280
    y1, dis = _tc1(npad, blk)(xg, fw, degp_t, W1)

    edge_agg = _sc_edge_agg(npad, ce, cpad)
    t1 = edge_agg(y1, src3, dst3)

    y2 = _tc2(npad, blk)(t1, y1, dis, b1[None, :], wcat)

    t2 = edge_agg(y2, src3, dst3)

    z = _tc3(npad, blk, dout)(t2, y2, dis, b_mu[None, :], b_ls[None, :],
                              noise_p)
    return z[:n]
